# baseline (speedup 1.0000x reference)
import functools

import jax
import jax.numpy as jnp
from jax import lax
from jax.experimental import pallas as pl
from jax.experimental.pallas import tpu as pltpu
from jax.experimental.pallas import tpu_sc as plsc

_NUM_CORES = 2
_NUM_SUBCORES = 16
_NUM_WORKERS = _NUM_CORES * _NUM_SUBCORES

_SG = 2


@functools.lru_cache(maxsize=None)
def _build(b: int, s: int, d: int):
    rw = b // _NUM_WORKERS
    n_grp = s // _SG
    assert rw * _NUM_WORKERS == b and rw == 128
    assert d == 64
    assert n_grp * _SG == s and n_grp % 2 == 0 and n_grp >= 6

    mesh = plsc.VectorSubcoreMesh(core_axis_name="c", subcore_axis_name="s")

    @functools.partial(
        pl.kernel,
        out_type=jax.ShapeDtypeStruct((s, d // 8, b // 128, 8, 128),
                                      jnp.float32),
        mesh=mesh,
        scratch_types=[
            pltpu.VMEM((rw, s), jnp.int32),
            pltpu.VMEM((s * rw,), jnp.int32),
            pltpu.VMEM((_SG, rw, d), jnp.float32),
            pltpu.VMEM((_SG, rw, d), jnp.float32),
            pltpu.VMEM((_SG, 8, 8, 129), jnp.float32),
            pltpu.VMEM((_SG, 8, 8, 129), jnp.float32),
            pltpu.SemaphoreType.DMA,
            pltpu.SemaphoreType.DMA,
            pltpu.SemaphoreType.DMA,
            pltpu.SemaphoreType.DMA,
        ],
        compiler_params=pltpu.CompilerParams(use_tc_tiling_on_sc=False,
                                             needs_layout_passes=False),
    )
    def gather_kernel(words_hbm, table_hbm, out5, idx_b, idx_t,
                      rows_a, rows_b, t_a, t_b, gs_a, gs_b, os_a, os_b):
        wid = lax.axis_index("s") * _NUM_CORES + lax.axis_index("c")
        pltpu.sync_copy(words_hbm.at[pl.ds(wid * rw, rw)], idx_b)

        iota = lax.iota(jnp.int32, 16)
        zeros16 = jnp.zeros((16,), jnp.int32)

        def t_idx(s_, carry):
            sv = zeros16 + s_
            for j in range(rw // 16):
                v = plsc.load_gather(idx_b, [iota + j * 16, sv])
                idx_t[pl.ds(s_ * rw + j * 16, 16)] = v
            return carry

        lax.fori_loop(0, s, t_idx, 0)

        def g_start(g, rows, sem):
            for j in range(_SG):
                pltpu.async_copy(
                    table_hbm.at[idx_t.at[pl.ds((g * _SG + j) * rw, rw)]],
                    rows.at[j], sem)

        def g_wait(g, rows, sem):
            for j in range(_SG):
                pltpu.make_async_copy(
                    table_hbm.at[idx_t.at[pl.ds((g * _SG + j) * rw, rw)]],
                    rows.at[j], sem).wait()

        def s_start(g, t, sem):
            pltpu.async_copy(t.at[:, :, :, pl.ds(0, 128)],
                             out5.at[pl.ds(g * _SG, _SG), :, wid], sem)

        def s_wait(g, t, sem):
            pltpu.make_async_copy(
                t.at[:, :, :, pl.ds(0, 128)],
                out5.at[pl.ds(g * _SG, _SG), :, wid], sem).wait()

        dr0 = iota // 8
        di0 = iota % 8

        def transpose(rows, t):
            @plsc.parallel_loop(0, rw, unroll=8)
            def tb(bi):
                bv = zeros16 + bi
                vs = [rows[sl, bi, pl.ds(d0, 16)]
                      for sl in range(_SG) for d0 in range(0, d, 16)]
                k = 0
                for sl in range(_SG):
                    for d0 in range(0, d, 16):
                        plsc.store_scatter(
                            t, [zeros16 + sl, dr0 + (d0 // 8), di0, bv],
                            vs[k])
                        k += 1

        n_pair = n_grp // 2

        g_start(0, rows_a, gs_a)
        g_start(1, rows_b, gs_b)

        def pair(i, carry):
            g0 = 2 * i
            g1 = g0 + 1

            @pl.when(i > 0)
            def _():
                s_wait(g0 - 2, t_a, os_a)

            g_wait(g0, rows_a, gs_a)
            transpose(rows_a, t_a)

            @pl.when(i < n_pair - 1)
            def _():
                g_start(g0 + 2, rows_a, gs_a)

            s_start(g0, t_a, os_a)

            @pl.when(i > 0)
            def _():
                s_wait(g1 - 2, t_b, os_b)

            g_wait(g1, rows_b, gs_b)
            transpose(rows_b, t_b)

            @pl.when(i < n_pair - 1)
            def _():
                g_start(g1 + 2, rows_b, gs_b)

            s_start(g1, t_b, os_b)
            return carry

        lax.fori_loop(0, n_pair, pair, 0)

        s_wait(n_grp - 2, t_a, os_a)
        s_wait(n_grp - 1, t_b, os_b)

    return gather_kernel


def kernel(words, table):
    b, s = words.shape
    _, d = table.shape
    out5 = _build(b, s, d)(words.astype(jnp.int32), table)
    return jnp.transpose(out5, (2, 4, 0, 1, 3)).reshape(b, s, d)


# speedup vs baseline: 1.0269x; 1.0269x over previous
import functools

import jax
import jax.numpy as jnp
from jax import lax
from jax.experimental import pallas as pl
from jax.experimental.pallas import tpu as pltpu
from jax.experimental.pallas import tpu_sc as plsc

_NUM_CORES = 2
_NUM_SUBCORES = 16
_NUM_WORKERS = _NUM_CORES * _NUM_SUBCORES

_SG = 2


@functools.lru_cache(maxsize=None)
def _build(b: int, s: int, d: int):
    rw = b // _NUM_WORKERS
    n_grp = s // _SG
    assert rw * _NUM_WORKERS == b and rw == 128
    assert d == 64
    assert n_grp * _SG == s and n_grp % 2 == 0 and n_grp >= 6

    mesh = plsc.VectorSubcoreMesh(core_axis_name="c", subcore_axis_name="s")

    @functools.partial(
        pl.kernel,
        out_type=jax.ShapeDtypeStruct((s, d // 8, b // 128, 8, 128),
                                      jnp.float32),
        mesh=mesh,
        scratch_types=[
            pltpu.VMEM((rw, s), jnp.int32),
            pltpu.VMEM((s * rw,), jnp.int32),
            pltpu.VMEM((_SG, rw, d), jnp.float32),
            pltpu.VMEM((_SG, rw, d), jnp.float32),
            pltpu.VMEM((_SG, 8, 8, 129), jnp.float32),
            pltpu.VMEM((_SG, 8, 8, 129), jnp.float32),
            pltpu.SemaphoreType.DMA,
            pltpu.SemaphoreType.DMA,
            pltpu.SemaphoreType.DMA,
            pltpu.SemaphoreType.DMA,
        ],
        compiler_params=pltpu.CompilerParams(use_tc_tiling_on_sc=False,
                                             needs_layout_passes=False),
    )
    def gather_kernel(words_hbm, table_hbm, out5, idx_b, idx_t,
                      rows_a, rows_b, t_a, t_b, gs_a, gs_b, os_a, os_b):
        wid = lax.axis_index("s") * _NUM_CORES + lax.axis_index("c")
        pltpu.sync_copy(words_hbm.at[pl.ds(wid * rw, rw)], idx_b)

        iota = lax.iota(jnp.int32, 16)
        zeros16 = jnp.zeros((16,), jnp.int32)

        def t_idx(s_, carry):
            sv = zeros16 + s_
            for j in range(rw // 16):
                v = plsc.load_gather(idx_b, [iota + j * 16, sv])
                idx_t[pl.ds(s_ * rw + j * 16, 16)] = v
            return carry

        lax.fori_loop(0, s, t_idx, 0)

        def g_start(g, rows, sem):
            for j in range(_SG):
                pltpu.async_copy(
                    table_hbm.at[idx_t.at[pl.ds((g * _SG + j) * rw, rw)]],
                    rows.at[j], sem)

        def g_wait(g, rows, sem):
            for j in range(_SG):
                pltpu.make_async_copy(
                    table_hbm.at[idx_t.at[pl.ds((g * _SG + j) * rw, rw)]],
                    rows.at[j], sem).wait()

        def s_start(g, t, sem):
            pltpu.async_copy(t.at[:, :, :, pl.ds(0, 128)],
                             out5.at[pl.ds(g * _SG, _SG), :, wid], sem)

        def s_wait(g, t, sem):
            pltpu.make_async_copy(
                t.at[:, :, :, pl.ds(0, 128)],
                out5.at[pl.ds(g * _SG, _SG), :, wid], sem).wait()

        dr0 = iota // 8
        di0 = iota % 8

        def transpose(rows, t):
            @plsc.parallel_loop(0, rw, unroll=4)
            def tb(bi):
                bv = zeros16 + bi
                vs = [rows[sl, bi, pl.ds(d0, 16)]
                      for sl in range(_SG) for d0 in range(0, d, 16)]
                k = 0
                for sl in range(_SG):
                    for d0 in range(0, d, 16):
                        plsc.store_scatter(
                            t, [zeros16 + sl, dr0 + (d0 // 8), di0, bv],
                            vs[k])
                        k += 1

        n_pair = n_grp // 2

        g_start(0, rows_a, gs_a)
        g_start(1, rows_b, gs_b)

        def pair(i, carry):
            g0 = 2 * i
            g1 = g0 + 1

            @pl.when(i > 0)
            def _():
                s_wait(g0 - 2, t_a, os_a)

            g_wait(g0, rows_a, gs_a)
            transpose(rows_a, t_a)

            @pl.when(i < n_pair - 1)
            def _():
                g_start(g0 + 2, rows_a, gs_a)

            s_start(g0, t_a, os_a)

            @pl.when(i > 0)
            def _():
                s_wait(g1 - 2, t_b, os_b)

            g_wait(g1, rows_b, gs_b)
            transpose(rows_b, t_b)

            @pl.when(i < n_pair - 1)
            def _():
                g_start(g1 + 2, rows_b, gs_b)

            s_start(g1, t_b, os_b)
            return carry

        lax.fori_loop(0, n_pair, pair, 0)

        s_wait(n_grp - 2, t_a, os_a)
        s_wait(n_grp - 1, t_b, os_b)

    return gather_kernel


def kernel(words, table):
    b, s = words.shape
    _, d = table.shape
    out5 = _build(b, s, d)(words.astype(jnp.int32), table)
    return jnp.transpose(out5, (2, 4, 0, 1, 3)).reshape(b, s, d)


# speedup vs baseline: 1.0398x; 1.0125x over previous
import functools

import jax
import jax.numpy as jnp
from jax import lax
from jax.experimental import pallas as pl
from jax.experimental.pallas import tpu as pltpu
from jax.experimental.pallas import tpu_sc as plsc

_NUM_CORES = 2
_NUM_SUBCORES = 16
_NUM_WORKERS = _NUM_CORES * _NUM_SUBCORES

_SG = 2


@functools.lru_cache(maxsize=None)
def _build(b: int, s: int, d: int):
    rw = b // _NUM_WORKERS
    n_grp = s // _SG
    assert rw * _NUM_WORKERS == b and rw == 128
    assert d == 64
    assert n_grp * _SG == s and n_grp % 2 == 0 and n_grp >= 6

    mesh = plsc.VectorSubcoreMesh(core_axis_name="c", subcore_axis_name="s")

    @functools.partial(
        pl.kernel,
        out_type=jax.ShapeDtypeStruct((s, d // 8, b // 128, 8, 128),
                                      jnp.float32),
        mesh=mesh,
        scratch_types=[
            pltpu.VMEM((rw, s), jnp.int32),
            pltpu.VMEM((s * rw,), jnp.int32),
            pltpu.VMEM((_SG, rw, d), jnp.float32),
            pltpu.VMEM((_SG, rw, d), jnp.float32),
            pltpu.VMEM((_SG, 8, 8, 129), jnp.float32),
            pltpu.VMEM((_SG, 8, 8, 129), jnp.float32),
            pltpu.SemaphoreType.DMA,
            pltpu.SemaphoreType.DMA,
            pltpu.SemaphoreType.DMA,
            pltpu.SemaphoreType.DMA,
        ],
        compiler_params=pltpu.CompilerParams(use_tc_tiling_on_sc=False,
                                             needs_layout_passes=False),
    )
    def gather_kernel(words_hbm, table_hbm, out5, idx_b, idx_t,
                      rows_a, rows_b, t_a, t_b, gs_a, gs_b, os_a, os_b):
        wid = lax.axis_index("s") * _NUM_CORES + lax.axis_index("c")
        pltpu.sync_copy(words_hbm.at[pl.ds(wid * rw, rw)], idx_b)

        iota = lax.iota(jnp.int32, 16)
        zeros16 = jnp.zeros((16,), jnp.int32)

        @plsc.parallel_loop(0, s, unroll=2)
        def t_idx(s_):
            sv = zeros16 + s_
            vs = [plsc.load_gather(idx_b, [iota + j * 16, sv])
                  for j in range(rw // 16)]
            for j in range(rw // 16):
                idx_t[pl.ds(s_ * rw + j * 16, 16)] = vs[j]

        def g_start(g, rows, sem):
            for j in range(_SG):
                pltpu.async_copy(
                    table_hbm.at[idx_t.at[pl.ds((g * _SG + j) * rw, rw)]],
                    rows.at[j], sem)

        def g_wait(g, rows, sem):
            for j in range(_SG):
                pltpu.make_async_copy(
                    table_hbm.at[idx_t.at[pl.ds((g * _SG + j) * rw, rw)]],
                    rows.at[j], sem).wait()

        def s_start(g, t, sem):
            pltpu.async_copy(t.at[:, :, :, pl.ds(0, 128)],
                             out5.at[pl.ds(g * _SG, _SG), :, wid], sem)

        def s_wait(g, t, sem):
            pltpu.make_async_copy(
                t.at[:, :, :, pl.ds(0, 128)],
                out5.at[pl.ds(g * _SG, _SG), :, wid], sem).wait()

        dr0 = iota // 8
        di0 = iota % 8

        def transpose(rows, t):
            @plsc.parallel_loop(0, rw, unroll=4)
            def tb(bi):
                bv = zeros16 + bi
                vs = [rows[sl, bi, pl.ds(d0, 16)]
                      for sl in range(_SG) for d0 in range(0, d, 16)]
                k = 0
                for sl in range(_SG):
                    for d0 in range(0, d, 16):
                        plsc.store_scatter(
                            t, [zeros16 + sl, dr0 + (d0 // 8), di0, bv],
                            vs[k])
                        k += 1

        n_pair = n_grp // 2

        g_start(0, rows_a, gs_a)
        g_start(1, rows_b, gs_b)

        def pair(i, carry):
            g0 = 2 * i
            g1 = g0 + 1

            @pl.when(i > 0)
            def _():
                s_wait(g0 - 2, t_a, os_a)

            g_wait(g0, rows_a, gs_a)
            transpose(rows_a, t_a)

            @pl.when(i < n_pair - 1)
            def _():
                g_start(g0 + 2, rows_a, gs_a)

            s_start(g0, t_a, os_a)

            @pl.when(i > 0)
            def _():
                s_wait(g1 - 2, t_b, os_b)

            g_wait(g1, rows_b, gs_b)
            transpose(rows_b, t_b)

            @pl.when(i < n_pair - 1)
            def _():
                g_start(g1 + 2, rows_b, gs_b)

            s_start(g1, t_b, os_b)
            return carry

        lax.fori_loop(0, n_pair, pair, 0)

        s_wait(n_grp - 2, t_a, os_a)
        s_wait(n_grp - 1, t_b, os_b)

    return gather_kernel


def kernel(words, table):
    b, s = words.shape
    _, d = table.shape
    out5 = _build(b, s, d)(words.astype(jnp.int32), table)
    return jnp.transpose(out5, (2, 4, 0, 1, 3)).reshape(b, s, d)
